# baseline (device time: 93213 ns/iter reference)
import jax
import jax.numpy as jnp
from jax import lax
from jax.experimental import pallas as pl
from jax.experimental.pallas import tpu as pltpu

N_DEV = 8
M = 1024
N_TOT = 4096
N_CHUNK = N_TOT // N_DEV
N_HALF = N_CHUNK // 2
N_SEG = 4
SEG_M = M // N_SEG


def kernel(x):
    def body(x_ref, out_ref, cw_ref, ccw_ref, cw_ssem, cw_rsem, ccw_ssem, ccw_rsem):
        p = lax.axis_index("i")
        left = jnp.mod(p - 1, N_DEV)
        right = jnp.mod(p + 1, N_DEV)

        barrier_sem = pltpu.get_barrier_semaphore()
        pl.semaphore_signal(
            barrier_sem, inc=1, device_id=(left,),
            device_id_type=pl.DeviceIdType.MESH,
        )
        pl.semaphore_signal(
            barrier_sem, inc=1, device_id=(right,),
            device_id_type=pl.DeviceIdType.MESH,
        )
        pl.semaphore_wait(barrier_sem, 2)

        def mk(d_ref, ssem, rsem, tgt, s, k):
            src = N_DEV - 1 if s == 0 else s - 1
            return pltpu.make_async_remote_copy(
                src_ref=d_ref.at[src, pl.ds(k * SEG_M, SEG_M)],
                dst_ref=d_ref.at[s, pl.ds(k * SEG_M, SEG_M)],
                send_sem=ssem.at[s, k],
                recv_sem=rsem.at[s, k],
                device_id=(tgt,),
                device_id_type=pl.DeviceIdType.MESH,
            )

        def mk_cw(s, k):
            return mk(cw_ref, cw_ssem, cw_rsem, right, s, k)

        def mk_ccw(s, k):
            return mk(ccw_ref, ccw_ssem, ccw_rsem, left, s, k)

        c_cw0 = jnp.mod(p - 1, N_DEV)
        c_ccw0 = jnp.mod(p + 1, N_DEV)
        cw_ref[N_DEV - 1] = x_ref[0, :, pl.ds(c_cw0 * N_CHUNK, N_HALF)]
        ccw_ref[N_DEV - 1] = x_ref[0, :, pl.ds(c_ccw0 * N_CHUNK + N_HALF, N_HALF)]
        for k in range(N_SEG):
            mk_cw(0, k).start()
            mk_ccw(0, k).start()

        for s in range(N_DEV - 1):
            c_cw = jnp.mod(p - 2 - s, N_DEV)
            c_ccw = jnp.mod(p + 2 + s, N_DEV)
            for k in range(N_SEG):
                rows = pl.ds(k * SEG_M, SEG_M)
                mk_cw(s, k).wait_recv()
                if s < N_DEV - 2:
                    cw_ref[s, rows] = cw_ref[s, rows] + x_ref[
                        0, rows, pl.ds(c_cw * N_CHUNK, N_HALF)
                    ]
                    mk_cw(s + 1, k).start()
                else:
                    out_ref[rows, 0:N_HALF] = cw_ref[s, rows] + x_ref[
                        0, rows, pl.ds(p * N_CHUNK, N_HALF)
                    ]
                mk_ccw(s, k).wait_recv()
                if s < N_DEV - 2:
                    ccw_ref[s, rows] = ccw_ref[s, rows] + x_ref[
                        0, rows, pl.ds(c_ccw * N_CHUNK + N_HALF, N_HALF)
                    ]
                    mk_ccw(s + 1, k).start()
                else:
                    out_ref[rows, N_HALF:N_CHUNK] = ccw_ref[s, rows] + x_ref[
                        0, rows, pl.ds(p * N_CHUNK + N_HALF, N_HALF)
                    ]

        for s in range(N_DEV - 1):
            for k in range(N_SEG):
                mk_cw(s, k).wait_send()
                mk_ccw(s, k).wait_send()

    return pl.pallas_call(
        body,
        out_shape=jax.ShapeDtypeStruct((M, N_CHUNK), jnp.float32),
        in_specs=[pl.BlockSpec(memory_space=pltpu.VMEM)],
        out_specs=pl.BlockSpec(memory_space=pltpu.VMEM),
        scratch_shapes=[
            pltpu.VMEM((N_DEV, M, N_HALF), jnp.float32),
            pltpu.VMEM((N_DEV, M, N_HALF), jnp.float32),
            pltpu.SemaphoreType.DMA((N_DEV - 1, N_SEG)),
            pltpu.SemaphoreType.DMA((N_DEV - 1, N_SEG)),
            pltpu.SemaphoreType.DMA((N_DEV - 1, N_SEG)),
            pltpu.SemaphoreType.DMA((N_DEV - 1, N_SEG)),
        ],
        compiler_params=pltpu.CompilerParams(collective_id=0),
    )(x)


# device time: 69715 ns/iter; 1.3371x vs baseline; 1.3371x over previous
import jax
import jax.numpy as jnp
from jax import lax
from jax.experimental import pallas as pl
from jax.experimental.pallas import tpu as pltpu

N_DEV = 8
M = 1024
N_TOT = 4096
N_CHUNK = N_TOT // N_DEV

ROW0 = (0, 344, 680)
ROWS = (344, 336, 344)


def _xor(a, b):
    return a + b - 2 * a * b


def _id3(vx, vy, vz):
    return 4 * vz + 2 * vy + _xor(vx, vy)


def kernel(x):
    def body(x_ref, out_ref, r1_0, r2_0, r3_0, r1_1, r2_1, r3_1, r1_2, r2_2,
             r3_2, ssems, rsems):
        p = lax.axis_index("i")
        z = p // 4
        pid = p % 4
        yb = pid // 2
        u = pid % 2
        mx = _xor(u, yb)
        my = yb
        mz = z

        qx = _id3(1 - mx, my, mz)
        qy = _id3(mx, 1 - my, mz)
        qz = _id3(mx, my, 1 - mz)

        barrier_sem = pltpu.get_barrier_semaphore()
        for q in (qx, qy, qz):
            pl.semaphore_signal(
                barrier_sem, inc=1, device_id=(q,),
                device_id_type=pl.DeviceIdType.MESH,
            )
        pl.semaphore_wait(barrier_sem, 3)

        flows = (
            dict(m=(mx, my, mz),
                 idf=lambda v1, v2, v3: _id3(v1, v2, v3),
                 q=(qx, qy, qz), r1=r1_0, r2=r2_0, r3=r3_0),
            dict(m=(my, mz, mx),
                 idf=lambda v1, v2, v3: _id3(v3, v1, v2),
                 q=(qy, qz, qx), r1=r1_1, r2=r2_1, r3=r3_1),
            dict(m=(mz, mx, my),
                 idf=lambda v1, v2, v3: _id3(v2, v3, v1),
                 q=(qz, qx, qy), r1=r1_2, r2=r2_2, r3=r3_2),
        )

        def xslice(f, j):
            return x_ref[0, pl.ds(ROW0[f], ROWS[f]), pl.ds(j * N_CHUNK, N_CHUNK)]

        def xsrc(f, j):
            return x_ref.at[0, pl.ds(ROW0[f], ROWS[f]), pl.ds(j * N_CHUNK, N_CHUNK)]

        def mk1(f, t2, t3):
            fl = flows[f]
            m1, m2, m3 = fl["m"]
            j = fl["idf"](1 - m1, _xor(t2, m2), _xor(t3, m3))
            return pltpu.make_async_remote_copy(
                src_ref=xsrc(f, j),
                dst_ref=fl["r1"].at[2 * t2 + t3],
                send_sem=ssems.at[f, 2 * t2 + t3],
                recv_sem=rsems.at[f, 2 * t2 + t3],
                device_id=(fl["q"][0],),
                device_id_type=pl.DeviceIdType.MESH,
            )

        def mk2(f, t3):
            fl = flows[f]
            return pltpu.make_async_remote_copy(
                src_ref=fl["r1"].at[2 + t3],
                dst_ref=fl["r2"].at[t3],
                send_sem=ssems.at[f, 4 + t3],
                recv_sem=rsems.at[f, 4 + t3],
                device_id=(fl["q"][1],),
                device_id_type=pl.DeviceIdType.MESH,
            )

        def mk3(f):
            fl = flows[f]
            return pltpu.make_async_remote_copy(
                src_ref=fl["r2"].at[1],
                dst_ref=fl["r3"],
                send_sem=ssems.at[f, 6],
                recv_sem=rsems.at[f, 6],
                device_id=(fl["q"][2],),
                device_id_type=pl.DeviceIdType.MESH,
            )

        for f in range(3):
            for t2 in range(2):
                for t3 in range(2):
                    mk1(f, t2, t3).start()

        for f in range(3):
            fl = flows[f]
            m1, m2, m3 = fl["m"]
            for t2 in range(2):
                for t3 in range(2):
                    mk1(f, t2, t3).wait_recv()
                    j = fl["idf"](m1, _xor(t2, m2), _xor(t3, m3))
                    s = 2 * t2 + t3
                    fl["r1"][s] = fl["r1"][s] + xslice(f, j)
            mk2(f, 0).start()
            mk2(f, 1).start()

        for f in range(3):
            fl = flows[f]
            for t3 in range(2):
                mk2(f, t3).wait_recv()
                fl["r2"][t3] = fl["r2"][t3] + fl["r1"][t3]
            mk3(f).start()

        for f in range(3):
            fl = flows[f]
            mk3(f).wait_recv()
            out_ref[pl.ds(ROW0[f], ROWS[f]), :] = fl["r2"][0] + fl["r3"][:, :]

        for f in range(3):
            for t2 in range(2):
                for t3 in range(2):
                    mk1(f, t2, t3).wait_send()
            mk2(f, 0).wait_send()
            mk2(f, 1).wait_send()
            mk3(f).wait_send()

    scratch = []
    for f in range(3):
        scratch.append(pltpu.VMEM((4, ROWS[f], N_CHUNK), jnp.float32))
        scratch.append(pltpu.VMEM((2, ROWS[f], N_CHUNK), jnp.float32))
        scratch.append(pltpu.VMEM((ROWS[f], N_CHUNK), jnp.float32))
    scratch.append(pltpu.SemaphoreType.DMA((3, 7)))
    scratch.append(pltpu.SemaphoreType.DMA((3, 7)))

    return pl.pallas_call(
        body,
        out_shape=jax.ShapeDtypeStruct((M, N_CHUNK), jnp.float32),
        in_specs=[pl.BlockSpec(memory_space=pltpu.VMEM)],
        out_specs=pl.BlockSpec(memory_space=pltpu.VMEM),
        scratch_shapes=scratch,
        compiler_params=pltpu.CompilerParams(collective_id=0),
    )(x)


# device time: 65504 ns/iter; 1.4230x vs baseline; 1.0643x over previous
import jax
import jax.numpy as jnp
from jax import lax
from jax.experimental import pallas as pl
from jax.experimental.pallas import tpu as pltpu

N_DEV = 8
M = 1024
N_TOT = 4096
N_CHUNK = N_TOT // N_DEV

ROW0 = (0, 344, 680)
ROWS = (344, 336, 344)


def _xor(a, b):
    return a + b - 2 * a * b


def _id3(vx, vy, vz):
    return 4 * vz + 2 * vy + _xor(vx, vy)


def kernel(x):
    def body(x_ref, out_ref, r1_0, r2_0, r3_0, r1_1, r2_1, r3_1, r1_2, r2_2,
             r3_2, ssems, rsems):
        p = lax.axis_index("i")
        z = p // 4
        pid = p % 4
        yb = pid // 2
        u = pid % 2
        mx = _xor(u, yb)
        my = yb
        mz = z

        qx = _id3(1 - mx, my, mz)
        qy = _id3(mx, 1 - my, mz)
        qz = _id3(mx, my, 1 - mz)

        barrier_sem = pltpu.get_barrier_semaphore()
        for q in (qx, qy, qz):
            pl.semaphore_signal(
                barrier_sem, inc=1, device_id=(q,),
                device_id_type=pl.DeviceIdType.MESH,
            )
        pl.semaphore_wait(barrier_sem, 3)

        flows = (
            dict(m=(mx, my, mz),
                 idf=lambda v1, v2, v3: _id3(v1, v2, v3),
                 q=(qx, qy, qz), r1=r1_0, r2=r2_0, r3=r3_0),
            dict(m=(my, mz, mx),
                 idf=lambda v1, v2, v3: _id3(v3, v1, v2),
                 q=(qy, qz, qx), r1=r1_1, r2=r2_1, r3=r3_1),
            dict(m=(mz, mx, my),
                 idf=lambda v1, v2, v3: _id3(v2, v3, v1),
                 q=(qz, qx, qy), r1=r1_2, r2=r2_2, r3=r3_2),
        )

        def xslice(f, j):
            return x_ref[0, pl.ds(ROW0[f], ROWS[f]), pl.ds(j * N_CHUNK, N_CHUNK)]

        def xsrc(f, j):
            return x_ref.at[0, pl.ds(ROW0[f], ROWS[f]), pl.ds(j * N_CHUNK, N_CHUNK)]

        def mk1(f, t2, t3):
            fl = flows[f]
            m1, m2, m3 = fl["m"]
            j = fl["idf"](1 - m1, _xor(t2, m2), _xor(t3, m3))
            return pltpu.make_async_remote_copy(
                src_ref=xsrc(f, j),
                dst_ref=fl["r1"].at[2 * t2 + t3],
                send_sem=ssems.at[f, 2 * t2 + t3],
                recv_sem=rsems.at[f, 2 * t2 + t3],
                device_id=(fl["q"][0],),
                device_id_type=pl.DeviceIdType.MESH,
            )

        def mk2(f, t3):
            fl = flows[f]
            return pltpu.make_async_remote_copy(
                src_ref=fl["r1"].at[2 + t3],
                dst_ref=fl["r2"].at[t3],
                send_sem=ssems.at[f, 4 + t3],
                recv_sem=rsems.at[f, 4 + t3],
                device_id=(fl["q"][1],),
                device_id_type=pl.DeviceIdType.MESH,
            )

        def mk3(f):
            fl = flows[f]
            return pltpu.make_async_remote_copy(
                src_ref=fl["r2"].at[1],
                dst_ref=fl["r3"],
                send_sem=ssems.at[f, 6],
                recv_sem=rsems.at[f, 6],
                device_id=(fl["q"][2],),
                device_id_type=pl.DeviceIdType.MESH,
            )

        ORDER = ((1, 1), (1, 0), (0, 1), (0, 0))

        def absorb1(f, t2, t3):
            fl = flows[f]
            m1, m2, m3 = fl["m"]
            mk1(f, t2, t3).wait_recv()
            j = fl["idf"](m1, _xor(t2, m2), _xor(t3, m3))
            s = 2 * t2 + t3
            fl["r1"][s] = fl["r1"][s] + xslice(f, j)

        for t2, t3 in ORDER:
            for f in range(3):
                mk1(f, t2, t3).start()

        for f in range(3):
            absorb1(f, 1, 1)
            mk2(f, 1).start()
        for f in range(3):
            absorb1(f, 1, 0)
            mk2(f, 0).start()
        for f in range(3):
            absorb1(f, 0, 1)
        for f in range(3):
            fl = flows[f]
            mk2(f, 1).wait_recv()
            fl["r2"][1] = fl["r2"][1] + fl["r1"][1]
            mk3(f).start()
        for f in range(3):
            absorb1(f, 0, 0)
        for f in range(3):
            fl = flows[f]
            mk2(f, 0).wait_recv()
            fl["r2"][0] = fl["r2"][0] + fl["r1"][0]
        for f in range(3):
            fl = flows[f]
            mk3(f).wait_recv()
            out_ref[pl.ds(ROW0[f], ROWS[f]), :] = fl["r2"][0] + fl["r3"][:, :]

        for f in range(3):
            for t2 in range(2):
                for t3 in range(2):
                    mk1(f, t2, t3).wait_send()
            mk2(f, 0).wait_send()
            mk2(f, 1).wait_send()
            mk3(f).wait_send()

    scratch = []
    for f in range(3):
        scratch.append(pltpu.VMEM((4, ROWS[f], N_CHUNK), jnp.float32))
        scratch.append(pltpu.VMEM((2, ROWS[f], N_CHUNK), jnp.float32))
        scratch.append(pltpu.VMEM((ROWS[f], N_CHUNK), jnp.float32))
    scratch.append(pltpu.SemaphoreType.DMA((3, 7)))
    scratch.append(pltpu.SemaphoreType.DMA((3, 7)))

    return pl.pallas_call(
        body,
        out_shape=jax.ShapeDtypeStruct((M, N_CHUNK), jnp.float32),
        in_specs=[pl.BlockSpec(memory_space=pltpu.VMEM)],
        out_specs=pl.BlockSpec(memory_space=pltpu.VMEM),
        scratch_shapes=scratch,
        compiler_params=pltpu.CompilerParams(collective_id=0),
    )(x)
